# baseline (speedup 1.0000x reference)
import functools

import jax
import jax.numpy as jnp
from jax import lax
from jax.experimental import pallas as pl
from jax.experimental.pallas import tpu as pltpu
from jax.experimental.pallas import tpu_sc as plsc

BATCH = 16384
EMBED_DIM = 64
OUT_DIM = 3 * EMBED_DIM + 1
MEMBERSHIP_VOCAB = 5
NUM_CORES = 2
NUM_SUBCORES = 16
NUM_WORKERS = NUM_CORES * NUM_SUBCORES
B_PER_W = BATCH // NUM_WORKERS
CHUNK = 128
NCHUNK = B_PER_W // CHUNK
LANES = 16
AGE_MEAN = 35.0
AGE_STD = 14.0

_mesh = plsc.VectorSubcoreMesh(
    core_axis_name="c", subcore_axis_name="s",
    num_cores=NUM_CORES, num_subcores=NUM_SUBCORES,
)


@functools.partial(
    pl.kernel,
    out_type=jax.ShapeDtypeStruct((BATCH, OUT_DIM), jnp.float32),
    mesh=_mesh,
    scratch_types=[
        pltpu.VMEM((CHUNK,), jnp.int32),
        pltpu.VMEM((CHUNK,), jnp.int32),
        pltpu.VMEM((CHUNK,), jnp.int32),
        pltpu.VMEM((CHUNK,), jnp.float32),
        pltpu.VMEM((CHUNK, EMBED_DIM), jnp.float32),
        pltpu.VMEM((CHUNK, EMBED_DIM), jnp.float32),
        pltpu.VMEM((MEMBERSHIP_VOCAB, EMBED_DIM), jnp.float32),
        pltpu.VMEM((CHUNK, OUT_DIM), jnp.float32),
        pltpu.SemaphoreType.DMA,
        pltpu.SemaphoreType.DMA,
    ],
)
def _embed_kernel(cust_hbm, club_hbm, post_hbm, age_hbm,
                  id_tab, mem_tab, zip_tab, out_hbm,
                  ids_v, post_v, club_v, age_v,
                  buf1_v, buf3_v, mem_v, asm_v, sem1, sem3):
    wid = lax.axis_index("s") * NUM_CORES + lax.axis_index("c")
    base = wid * B_PER_W
    inv_std = jnp.float32(1.0 / AGE_STD)

    pltpu.sync_copy(mem_tab, mem_v)

    def chunk_body(k, carry):
        off = base + k * CHUNK
        pltpu.sync_copy(cust_hbm.at[pl.ds(off, CHUNK)], ids_v)
        pltpu.sync_copy(post_hbm.at[pl.ds(off, CHUNK)], post_v)
        pltpu.sync_copy(club_hbm.at[pl.ds(off, CHUNK)], club_v)
        pltpu.sync_copy(age_hbm.at[pl.ds(off, CHUNK)], age_v)

        def fire(g, carry2):
            iv1 = ids_v[pl.ds(g * LANES, LANES)]
            iv3 = post_v[pl.ds(g * LANES, LANES)]
            for i in range(LANES):
                r = g * LANES + i
                pltpu.make_async_copy(
                    id_tab.at[iv1[i]], buf1_v.at[r], sem1).start()
                pltpu.make_async_copy(
                    zip_tab.at[iv3[i]], buf3_v.at[r], sem3).start()
            return carry2

        lax.fori_loop(0, CHUNK // LANES, fire, 0)

        def drain(r, carry2):
            pltpu.make_async_copy(
                id_tab.at[0], buf1_v.at[0], sem1).wait()
            pltpu.make_async_copy(
                zip_tab.at[0], buf3_v.at[0], sem3).wait()
            return carry2

        lax.fori_loop(0, CHUNK, drain, 0)

        def group_body(g, carry2):
            a16 = (age_v[pl.ds(g * LANES, LANES)] - AGE_MEAN) * inv_std
            c16 = club_v[pl.ds(g * LANES, LANES)]
            for i in range(LANES):
                r = g * LANES + i
                m = c16[i]
                asm_v[r, pl.ds(OUT_DIM - LANES, LANES)] = jnp.broadcast_to(
                    a16[i], (LANES,))
                for c in range(EMBED_DIM // LANES):
                    s = pl.ds(c * LANES, LANES)
                    asm_v[r, pl.ds(c * LANES, LANES)] = buf1_v[r, s]
                    asm_v[r, pl.ds(EMBED_DIM + c * LANES, LANES)] = mem_v[m, s]
                    asm_v[r, pl.ds(2 * EMBED_DIM + c * LANES, LANES)] = \
                        buf3_v[r, s]
            return carry2

        lax.fori_loop(0, CHUNK // LANES, group_body, 0)

        pltpu.sync_copy(asm_v, out_hbm.at[pl.ds(off, CHUNK)])
        return carry

    lax.fori_loop(0, NCHUNK, chunk_body, 0)


def kernel(customer_id, club_member_status, postal_code, age,
           id_table, membership_table, zip_table):
    return _embed_kernel(
        customer_id.astype(jnp.int32),
        club_member_status.astype(jnp.int32),
        postal_code.astype(jnp.int32),
        age.astype(jnp.float32),
        id_table, membership_table, zip_table,
    )
